# baseline (speedup 1.0000x reference)
import functools

import jax
import jax.numpy as jnp
from jax import lax
from jax.experimental import pallas as pl
from jax.experimental.pallas import tpu as pltpu
from jax.experimental.pallas import tpu_sc as plsc

B = 128
N = 32768
KTOP = 8
L = 16
NCHUNK = N // L
NSTRIP = 16
CPS = NCHUNK // NSTRIP
UN = 16
NW = 32
ROWS_PER = B // NW

NEG = float("-inf")
BIG = 1 << 30


def _tree_max(vs):
    while len(vs) > 1:
        vs = [jnp.maximum(a, b) for a, b in zip(vs[::2], vs[1::2])]
    return vs[0]


SUBC = 16
NSUB = CPS // SUBC


def _topk_rows(w_hbm, out_hbm, rowbuf, outbuf, subs, semi, semo):
    cid = lax.axis_index("c")
    sid = lax.axis_index("s")
    wid = sid * 2 + cid
    lanes = lax.iota(jnp.int32, L)
    zeros16 = jnp.zeros((L,), jnp.float32)
    neg16 = jnp.full((L,), NEG, jnp.float32)
    big16 = jnp.full((L,), BIG, jnp.int32)
    zi16 = jnp.zeros((L,), jnp.int32)
    lane0 = lanes == 0
    sel8 = lanes < KTOP

    base_row = wid * ROWS_PER
    pltpu.async_copy(w_hbm.at[base_row], rowbuf.at[pl.ds(0, N)], semi)

    @plsc.parallel_loop(0, NCHUNK // UN)
    def _(i):
        for u in range(UN):
            outbuf[pl.ds((i * UN + u) * L, L)] = zeros16

    def process(roff, row, idx_prev):
        def strip_body(j, smax16):
            def sub_body(s, mx):
                base = roff + (j * CPS + s * SUBC) * L
                vs = [rowbuf[pl.ds(base + u * L, L)] for u in range(SUBC)]
                msub = _tree_max(vs)
                subs[pl.ds((j * NSUB + s) * L, L)] = msub
                return jnp.maximum(mx, msub)
            mx = plsc.parallel_loop(0, NSUB, carry=neg16, unroll=2)(sub_body)
            return jnp.where(lanes == j, jnp.max(mx), smax16)
        smax16 = lax.fori_loop(0, NSTRIP, strip_body, neg16)

        def ex_body(it, carry):
            vals8, idx8, smax16 = carry
            gmax = jnp.max(smax16)
            minj = jnp.min(jnp.where(smax16 == gmax, lanes, jnp.int32(99)))

            def sub_find(s, ms):
                sub = subs[pl.ds((minj * NSUB + s) * L, L)]
                return jnp.minimum(ms, jnp.where(sub == gmax, s, 99))
            ms = lax.fori_loop(0, NSUB, sub_find, jnp.full((L,), 99, jnp.int32))
            mins = jnp.min(ms)

            sbase = (minj * CPS + mins * SUBC) * L
            midx, cnt, mlt = big16, zi16, neg16
            for u in range(SUBC):
                v = rowbuf[pl.ds(roff + sbase + u * L, L)]
                eq = v == gmax
                midx = jnp.minimum(midx, jnp.where(eq, sbase + u * L + lanes, BIG))
                cnt = cnt + eq.astype(jnp.int32)
                mlt = jnp.maximum(mlt, jnp.where(eq, NEG, v))
            idx = jnp.min(midx)

            plsc.store_scatter(rowbuf, [jnp.full((L,), roff + idx)], neg16, mask=lane0)
            cnt_adj = cnt - (lanes == (idx & (L - 1))).astype(jnp.int32)
            newslice = jnp.where(cnt_adj > 0, gmax, mlt)
            subs[pl.ds((minj * NSUB + mins) * L, L)] = newslice

            def strip_max(s, mx):
                return jnp.maximum(mx, subs[pl.ds((minj * NSUB + s) * L, L)])
            mstrip = lax.fori_loop(0, NSUB, strip_max, neg16)
            smax16 = jnp.where(lanes == minj, jnp.max(mstrip), smax16)

            vals8 = jnp.where(lanes == it, gmax, vals8)
            idx8 = jnp.where(lanes == it, idx, idx8)
            return vals8, idx8, smax16

        vals8, idx8, _ = lax.fori_loop(
            0, KTOP, ex_body, (zeros16, zi16, smax16))

        l1 = jnp.sum(jnp.where(sel8, jnp.abs(vals8), 0.0))
        invv = 1.0 / jnp.maximum(jnp.full((L,), l1), jnp.float32(1e-12))

        @pl.when(row > base_row)
        def _():
            pltpu.make_async_copy(outbuf, out_hbm.at[row - 1], semo).wait()
            plsc.store_scatter(outbuf, [idx_prev], zeros16, mask=sel8)
        plsc.store_scatter(outbuf, [idx8], vals8 * invv, mask=sel8)
        pltpu.async_copy(outbuf, out_hbm.at[row], semo)
        return idx8

    def row_body(r, idx_prev):
        row = base_row + r
        roff = (r & 1) * N
        pltpu.make_async_copy(
            w_hbm.at[row], rowbuf.at[pl.ds(roff, N)], semi).wait()

        @pl.when(r < ROWS_PER - 1)
        def _():
            pltpu.async_copy(
                w_hbm.at[row + 1], rowbuf.at[pl.ds(N - roff, N)], semi)

        return process(roff, row, idx_prev)

    lax.fori_loop(0, ROWS_PER, row_body, zi16)
    pltpu.make_async_copy(outbuf, out_hbm.at[base_row + ROWS_PER - 1], semo).wait()


def kernel(weights, k):
    del k
    mesh = plsc.VectorSubcoreMesh(core_axis_name="c", subcore_axis_name="s")
    run = functools.partial(
        pl.kernel,
        mesh=mesh,
        compiler_params=pltpu.CompilerParams(needs_layout_passes=False),
        out_type=jax.ShapeDtypeStruct((B, N), jnp.float32),
        scratch_types=[
            pltpu.VMEM((2 * N,), jnp.float32),
            pltpu.VMEM((N,), jnp.float32),
            pltpu.VMEM((NSTRIP * NSUB * L,), jnp.float32),
            pltpu.SemaphoreType.DMA,
            pltpu.SemaphoreType.DMA,
        ],
    )(_topk_rows)
    return run(weights)


# speedup vs baseline: 1.0065x; 1.0065x over previous
import functools

import jax
import jax.numpy as jnp
from jax import lax
from jax.experimental import pallas as pl
from jax.experimental.pallas import tpu as pltpu
from jax.experimental.pallas import tpu_sc as plsc

B = 128
N = 32768
KTOP = 8
L = 16
NCHUNK = N // L
NSTRIP = 16
CPS = NCHUNK // NSTRIP
UN = 16
NW = 32
ROWS_PER = B // NW

NEG = float("-inf")
BIG = 1 << 30


def _tree_max(vs):
    while len(vs) > 1:
        vs = [jnp.maximum(a, b) for a, b in zip(vs[::2], vs[1::2])]
    return vs[0]


SUBC = 16
NSUB = CPS // SUBC


def _topk_rows(w_hbm, out_hbm, rowbuf, outbuf, subs, semi, semo):
    cid = lax.axis_index("c")
    sid = lax.axis_index("s")
    wid = sid * 2 + cid
    lanes = lax.iota(jnp.int32, L)
    zeros16 = jnp.zeros((L,), jnp.float32)
    neg16 = jnp.full((L,), NEG, jnp.float32)
    big16 = jnp.full((L,), BIG, jnp.int32)
    zi16 = jnp.zeros((L,), jnp.int32)
    lane0 = lanes == 0
    sel8 = lanes < KTOP

    base_row = wid * ROWS_PER
    pltpu.async_copy(w_hbm.at[base_row], rowbuf.at[pl.ds(0, N)], semi)

    @plsc.parallel_loop(0, NCHUNK // UN)
    def _(i):
        for u in range(UN):
            outbuf[pl.ds((i * UN + u) * L, L)] = zeros16

    def process(roff, row, idx_prev):
        def strip_body(j, smax16):
            def sub_body(s, mx):
                base = roff + (j * CPS + s * SUBC) * L
                vs = [rowbuf[pl.ds(base + u * L, L)] for u in range(SUBC)]
                msub = _tree_max(vs)
                subs[pl.ds((j * NSUB + s) * L, L)] = msub
                return jnp.maximum(mx, msub)
            mx = plsc.parallel_loop(0, NSUB, carry=neg16)(sub_body)
            return jnp.where(lanes == j, jnp.max(mx), smax16)
        smax16 = lax.fori_loop(0, NSTRIP, strip_body, neg16)

        def ex_body(it, carry):
            vals8, idx8, smax16 = carry
            gmax = jnp.max(smax16)
            minj = jnp.min(jnp.where(smax16 == gmax, lanes, jnp.int32(99)))

            def sub_find(s, ms):
                sub = subs[pl.ds((minj * NSUB + s) * L, L)]
                return jnp.minimum(ms, jnp.where(sub == gmax, s, 99))
            ms = lax.fori_loop(0, NSUB, sub_find, jnp.full((L,), 99, jnp.int32))
            mins = jnp.min(ms)

            sbase = (minj * CPS + mins * SUBC) * L
            midx, cnt, mlt = big16, zi16, neg16
            for u in range(SUBC):
                v = rowbuf[pl.ds(roff + sbase + u * L, L)]
                eq = v == gmax
                midx = jnp.minimum(midx, jnp.where(eq, sbase + u * L + lanes, BIG))
                cnt = cnt + eq.astype(jnp.int32)
                mlt = jnp.maximum(mlt, jnp.where(eq, NEG, v))
            idx = jnp.min(midx)

            plsc.store_scatter(rowbuf, [jnp.full((L,), roff + idx)], neg16, mask=lane0)
            cnt_adj = cnt - (lanes == (idx & (L - 1))).astype(jnp.int32)
            newslice = jnp.where(cnt_adj > 0, gmax, mlt)
            subs[pl.ds((minj * NSUB + mins) * L, L)] = newslice

            def strip_max(s, mx):
                return jnp.maximum(mx, subs[pl.ds((minj * NSUB + s) * L, L)])
            mstrip = lax.fori_loop(0, NSUB, strip_max, neg16)
            smax16 = jnp.where(lanes == minj, jnp.max(mstrip), smax16)

            vals8 = jnp.where(lanes == it, gmax, vals8)
            idx8 = jnp.where(lanes == it, idx, idx8)
            return vals8, idx8, smax16

        vals8, idx8, _ = lax.fori_loop(
            0, KTOP, ex_body, (zeros16, zi16, smax16))

        l1 = jnp.sum(jnp.where(sel8, jnp.abs(vals8), 0.0))
        invv = 1.0 / jnp.maximum(jnp.full((L,), l1), jnp.float32(1e-12))

        @pl.when(row > base_row)
        def _():
            pltpu.make_async_copy(outbuf, out_hbm.at[row - 1], semo).wait()
            plsc.store_scatter(outbuf, [idx_prev], zeros16, mask=sel8)
        plsc.store_scatter(outbuf, [idx8], vals8 * invv, mask=sel8)
        pltpu.async_copy(outbuf, out_hbm.at[row], semo)
        return idx8

    def row_body(r, idx_prev):
        row = base_row + r
        roff = (r & 1) * N
        pltpu.make_async_copy(
            w_hbm.at[row], rowbuf.at[pl.ds(roff, N)], semi).wait()

        @pl.when(r < ROWS_PER - 1)
        def _():
            pltpu.async_copy(
                w_hbm.at[row + 1], rowbuf.at[pl.ds(N - roff, N)], semi)

        return process(roff, row, idx_prev)

    lax.fori_loop(0, ROWS_PER, row_body, zi16)
    pltpu.make_async_copy(outbuf, out_hbm.at[base_row + ROWS_PER - 1], semo).wait()


def kernel(weights, k):
    del k
    mesh = plsc.VectorSubcoreMesh(core_axis_name="c", subcore_axis_name="s")
    run = functools.partial(
        pl.kernel,
        mesh=mesh,
        compiler_params=pltpu.CompilerParams(needs_layout_passes=False),
        out_type=jax.ShapeDtypeStruct((B, N), jnp.float32),
        scratch_types=[
            pltpu.VMEM((2 * N,), jnp.float32),
            pltpu.VMEM((N,), jnp.float32),
            pltpu.VMEM((NSTRIP * NSUB * L,), jnp.float32),
            pltpu.SemaphoreType.DMA,
            pltpu.SemaphoreType.DMA,
        ],
    )(_topk_rows)
    return run(weights)
